# baseline (speedup 1.0000x reference)
import functools

import jax
import jax.numpy as jnp
from jax import lax
from jax.experimental import pallas as pl
from jax.experimental.pallas import tpu as pltpu
from jax.experimental.pallas import tpu_sc as plsc

_NC = 2
_NS = 16
_L = 16

_K = 128
_PH = 2


def _propagate(hp, spair3, dpair3, wenc3, n_chunk, np_rows):
    _, npr, d = hp.shape
    dh = d // 2
    rows_per_tile = npr // _NS
    half = n_chunk // _PH
    mesh = plsc.VectorSubcoreMesh(core_axis_name="c", subcore_axis_name="s")

    @functools.partial(
        pl.kernel,
        mesh=mesh,
        out_type=jax.ShapeDtypeStruct((_NC, npr, d), jnp.float32),
        scratch_types=[
            pltpu.VMEM((half, _K), jnp.int32),
            pltpu.VMEM((half, _K), jnp.int32),
            pltpu.VMEM((half, _K), jnp.float32),
            pltpu.VMEM((_K, d), jnp.float32),
            pltpu.VMEM_SHARED((npr, d), jnp.float32),
            pltpu.VMEM_SHARED((npr, d), jnp.float32),
            pltpu.SemaphoreType.DMA,
        ],
    )
    def k(hp_hbm, src_hbm, dst_hbm, w_hbm, out_hbm,
          srcb, dstb, wb, rows, hc, acc, sem):
        c = lax.axis_index("c")
        s = lax.axis_index("s")
        base_row = s * rows_per_tile

        pltpu.sync_copy(hp_hbm.at[c, pl.ds(base_row, rows_per_tile)],
                        hc.at[pl.ds(base_row, rows_per_tile)])

        @pl.loop(0, _K)
        def _(i):
            for j in range(d // _L):
                rows[i, pl.ds(j * _L, _L)] = jnp.zeros((_L,), jnp.float32)

        for p in range(rows_per_tile // _K):
            pltpu.sync_copy(rows, acc.at[pl.ds(base_row + p * _K, _K)])
        ztail = rows_per_tile // _K * _K
        if ztail != rows_per_tile:
            pltpu.sync_copy(rows.at[pl.ds(0, rows_per_tile - ztail)],
                            acc.at[pl.ds(base_row + ztail,
                                         rows_per_tile - ztail)])

        plsc.subcore_barrier()

        for ph in range(_PH):
            pltpu.sync_copy(src_hbm.at[s, ph], srcb)
            pltpu.sync_copy(dst_hbm.at[s, ph], dstb)
            pltpu.sync_copy(w_hbm.at[s, ph], wb)

            @pl.loop(0, half)
            def _(ci):
                pltpu.async_copy(hc.at[srcb.at[ci]], rows, sem).wait()

                @pl.loop(0, _K // _L)
                def _(g):
                    wraw = wb[ci, pl.ds(g * _L, _L)]
                    soff = jnp.where(wraw < 0.0, dh, 0)
                    wa = jnp.abs(wraw)
                    doff = jnp.where(wa >= 2.0, dh, 0)
                    wsc = jnp.where(wa >= 2.0, wa - 2.0, wa)
                    zed = jnp.zeros((_L,), jnp.float32)
                    for i in range(_L):
                        e = g * _L + i
                        so = soff[i]
                        do = doff[i]
                        wv = jnp.full((_L,), wsc[i], dtype=jnp.float32)
                        for j in range(dh // _L):
                            val = rows[e, pl.ds(so + j * _L, _L)] * wv
                            rows[e, pl.ds(do + j * _L, _L)] = val
                        zo = dh - do
                        for j in range(dh // _L):
                            rows[e, pl.ds(zo + j * _L, _L)] = zed

                pltpu.sync_copy(rows, acc.at[dstb.at[ci]], add=True)

        plsc.subcore_barrier()
        pltpu.sync_copy(acc.at[pl.ds(base_row, rows_per_tile)],
                        out_hbm.at[c, pl.ds(base_row, rows_per_tile)])

    return k(hp, spair3, dpair3, wenc3)


def _final(x, h1, h2, h3, fw2, mapping):
    n, d = x.shape
    c_out = mapping.shape[1]

    def body(x_ref, h1_ref, h2_ref, h3_ref, fw_ref, map_ref,
             res_ref, fp_ref):
        fwv = fw_ref[...]
        m = jnp.max(fwv)
        e = jnp.exp(fwv - m)
        fp = e / jnp.sum(e)
        fp_ref[...] = fp

        feats = (x_ref[...], h1_ref[...], h2_ref[...], h3_ref[...])
        u = jnp.zeros((n, d), dtype=jnp.float32)
        for i, f in enumerate(feats):
            mu = jnp.mean(f, axis=0, keepdims=True)
            xc = f - mu
            var = jnp.sum(xc * xc, axis=0, keepdims=True) / (n - 1)
            std = jnp.sqrt(var)
            u = u + fp[0, i] * (xc / (std + 1e-6))
        u = jnp.maximum(u, 0.0)
        logits = jnp.dot(u, map_ref[...], preferred_element_type=jnp.float32)
        lmax = jnp.max(logits, axis=1, keepdims=True)
        ls = logits - lmax
        lse = jnp.log(jnp.sum(jnp.exp(ls), axis=1, keepdims=True))
        res_ref[...] = ls - lse

    return pl.pallas_call(
        body,
        out_shape=(
            jax.ShapeDtypeStruct((n, c_out), jnp.float32),
            jax.ShapeDtypeStruct((1, 4), jnp.float32),
        ),
    )(x, h1, h2, h3, fw2, mapping)


def kernel(x, edge_index, edge_weight, fw, mapping):
    n, d = x.shape
    e = edge_weight.shape[0]

    np_rows = -(-(n // 2) // (8 * _NS)) * 8 * _NS

    blk = _K * _PH
    ept = -(-e // (_NS * blk)) * blk
    pad = ept * _NS - e
    src = jnp.concatenate([edge_index[0], jnp.zeros((pad,), jnp.int32)])
    dst = jnp.concatenate([edge_index[1], jnp.zeros((pad,), jnp.int32)])
    w = jnp.concatenate([edge_weight, jnp.zeros((pad,), jnp.float32)])

    wenc = jnp.where(src % 2 == 1, -1.0, 1.0) * (w + 2.0 * (dst % 2))
    n_chunk = ept // _K
    shp = (_NS, _PH, n_chunk // _PH, _K)
    spair3 = (src // 2).astype(jnp.int32).reshape(shp)
    dpair3 = (dst // 2).astype(jnp.int32).reshape(shp)
    wenc3 = wenc.astype(jnp.float32).reshape(shp)

    dh = d // 2

    def pack(hmat):
        hpad = jnp.concatenate(
            [hmat, jnp.zeros((2 * np_rows - n, d), jnp.float32)], axis=0)
        h4 = hpad.reshape(np_rows, 2, 2, dh)
        return h4.transpose(2, 0, 1, 3).reshape(2, np_rows, d)

    def unpack(hpk):
        h4 = hpk.reshape(2, np_rows, 2, dh)
        return h4.transpose(1, 2, 0, 3).reshape(2 * np_rows, d)[:n]

    hp0 = pack(x)
    hp1 = _propagate(hp0, spair3, dpair3, wenc3, n_chunk, np_rows)
    hp2 = _propagate(hp1, spair3, dpair3, wenc3, n_chunk, np_rows)
    hp3 = _propagate(hp2, spair3, dpair3, wenc3, n_chunk, np_rows)

    res, fp2 = _final(x, unpack(hp1), unpack(hp2), unpack(hp3),
                      fw.reshape(1, 4), mapping)
    return res, fp2.reshape(4), 0


# speedup vs baseline: 1.5632x; 1.5632x over previous
import functools

import jax
import jax.numpy as jnp
from jax import lax
from jax.experimental import pallas as pl
from jax.experimental.pallas import tpu as pltpu
from jax.experimental.pallas import tpu_sc as plsc

_NC = 2
_NS = 16
_L = 16
_NW = _NC * _NS

_K = 128
_ZR = 208


def _propagate(h, src3, dst3, w3, n_chunk):
    n, d = h.shape
    rows_per_tile = (n // _NS) // 8 * 8
    rem = n - rows_per_tile * _NS
    mesh = plsc.VectorSubcoreMesh(core_axis_name="c", subcore_axis_name="s")

    @functools.partial(
        pl.kernel,
        mesh=mesh,
        out_type=jax.ShapeDtypeStruct((_NC, n, d), jnp.float32),
        scratch_types=[
            pltpu.VMEM((n_chunk // 2, _K), jnp.int32),
            pltpu.VMEM((n_chunk // 2, _K), jnp.int32),
            pltpu.VMEM((n_chunk // 2, _K), jnp.float32),
            pltpu.VMEM((_K, d), jnp.float32),
            pltpu.VMEM((_K, d), jnp.float32),
            pltpu.VMEM_SHARED((n, d), jnp.float32),
            pltpu.SemaphoreType.DMA,
            pltpu.SemaphoreType.DMA,
        ],
    )
    def k(h_hbm, src_hbm, dst_hbm, w_hbm, out_hbm,
          srcb, dstb, wb, rows, rows2, acc, sem, sem2):
        c = lax.axis_index("c")
        s = lax.axis_index("s")
        wid = s * _NC + c

        @pl.loop(0, _K)
        def _(i):
            for j in range(d // _L):
                rows[i, pl.ds(j * _L, _L)] = jnp.zeros((_L,), jnp.float32)

        base_row = s * rows_per_tile
        for p in range(rows_per_tile // _K):
            pltpu.sync_copy(rows, acc.at[pl.ds(base_row + p * _K, _K)])
        ztail = rows_per_tile // _K * _K
        pltpu.sync_copy(rows.at[pl.ds(0, rows_per_tile - ztail)],
                        acc.at[pl.ds(base_row + ztail, rows_per_tile - ztail)])

        @pl.when(s == _NS - 1)
        def _():
            pltpu.sync_copy(rows.at[pl.ds(0, rem)],
                            acc.at[pl.ds(_NS * rows_per_tile, rem)])

        plsc.subcore_barrier()

        def scale_scatter(rows_b, ci):
            @pl.loop(0, _K // _L)
            def _(g):
                wvec = wb[ci, pl.ds(g * _L, _L)]
                for i in range(_L):
                    wv = jnp.full((_L,), wvec[i], dtype=jnp.float32)
                    for j in range(d // _L):
                        sl = (g * _L + i, pl.ds(j * _L, _L))
                        rows_b[sl] = rows_b[sl] * wv

            pltpu.sync_copy(rows_b, acc.at[dstb.at[ci]], add=True)

        half = n_chunk // 2
        for ph in range(2):
            pltpu.sync_copy(src_hbm.at[wid, ph], srcb)
            pltpu.sync_copy(dst_hbm.at[wid, ph], dstb)
            pltpu.sync_copy(w_hbm.at[wid, ph], wb)

            pltpu.async_copy(h_hbm.at[srcb.at[0]], rows, sem).wait()

            @pl.loop(0, half // 2)
            def _(blk):
                i = 2 * blk
                hb = pltpu.async_copy(h_hbm.at[srcb.at[i + 1]], rows2, sem2)
                scale_scatter(rows, i)
                hb.wait()

                @pl.when(i + 2 < half)
                def _():
                    ha = pltpu.async_copy(h_hbm.at[srcb.at[i + 2]],
                                          rows, sem)
                    scale_scatter(rows2, i + 1)
                    ha.wait()

                @pl.when(i + 2 >= half)
                def _():
                    scale_scatter(rows2, i + 1)

        plsc.subcore_barrier()
        pltpu.sync_copy(acc.at[pl.ds(base_row, rows_per_tile)],
                        out_hbm.at[c, pl.ds(base_row, rows_per_tile)])

        @pl.when(s == _NS - 1)
        def _():
            tail = _NS * rows_per_tile
            pltpu.sync_copy(acc.at[pl.ds(tail, rem)],
                            out_hbm.at[c, pl.ds(tail, rem)])

    return k(h, src3, dst3, w3)


def _combine(parts):
    nc, n, d = parts.shape

    def body(p_ref, o_ref):
        o_ref[...] = p_ref[0] + p_ref[1]

    return pl.pallas_call(
        body,
        out_shape=jax.ShapeDtypeStruct((n, d), jnp.float32),
    )(parts)


def _final(x, h1, h2, h3, fw2, mapping):
    n, d = x.shape
    c = mapping.shape[1]

    def body(x_ref, h1_ref, h2_ref, h3_ref, fw_ref, map_ref,
             res_ref, fp_ref):
        fwv = fw_ref[...]
        m = jnp.max(fwv)
        e = jnp.exp(fwv - m)
        fp = e / jnp.sum(e)
        fp_ref[...] = fp

        u = jnp.zeros((n, d), dtype=jnp.float32)
        for i, ref in enumerate((x_ref, h1_ref, h2_ref, h3_ref)):
            f = ref[...]
            mu = jnp.mean(f, axis=0, keepdims=True)
            xc = f - mu
            var = jnp.sum(xc * xc, axis=0, keepdims=True) / (n - 1)
            std = jnp.sqrt(var)
            u = u + fp[0, i] * (xc / (std + 1e-6))
        u = jnp.maximum(u, 0.0)
        logits = jnp.dot(u, map_ref[...], preferred_element_type=jnp.float32)
        lmax = jnp.max(logits, axis=1, keepdims=True)
        ls = logits - lmax
        lse = jnp.log(jnp.sum(jnp.exp(ls), axis=1, keepdims=True))
        res_ref[...] = ls - lse

    return pl.pallas_call(
        body,
        out_shape=(
            jax.ShapeDtypeStruct((n, c), jnp.float32),
            jax.ShapeDtypeStruct((1, 4), jnp.float32),
        ),
    )(x, h1, h2, h3, fw2, mapping)


def kernel(x, edge_index, edge_weight, fw, mapping):
    n, d = x.shape
    e = edge_weight.shape[0]

    epw = -(-e // (_NW * _K * 4)) * _K * 4
    e_pad = epw * _NW
    pad = e_pad - e
    src = jnp.concatenate([edge_index[0], jnp.zeros((pad,), jnp.int32)])
    dst = jnp.concatenate([edge_index[1], jnp.zeros((pad,), jnp.int32)])
    w = jnp.concatenate([edge_weight, jnp.zeros((pad,), jnp.float32)])
    n_chunk = epw // _K
    src3 = src.reshape(_NW, 2, n_chunk // 2, _K)
    dst3 = dst.reshape(_NW, 2, n_chunk // 2, _K)
    w3 = w.reshape(_NW, 2, n_chunk // 2, _K)

    h1 = _combine(_propagate(x, src3, dst3, w3, n_chunk))
    h2 = _combine(_propagate(h1, src3, dst3, w3, n_chunk))
    h3 = _combine(_propagate(h2, src3, dst3, w3, n_chunk))

    res, fp2 = _final(x, h1, h2, h3, fw.reshape(1, 4), mapping)
    return res, fp2.reshape(4), 0


# speedup vs baseline: 1.9115x; 1.2228x over previous
import functools

import jax
import jax.numpy as jnp
from jax import lax
from jax.experimental import pallas as pl
from jax.experimental.pallas import tpu as pltpu
from jax.experimental.pallas import tpu_sc as plsc

_NC = 2
_NS = 16
_L = 16
_NW = _NC * _NS

_K = 128
_ZR = 208


def _propagate(h, src3, dst3, w3, n_chunk):
    n, d = h.shape
    rows_per_tile = (n // _NS) // 8 * 8
    rem = n - rows_per_tile * _NS
    mesh = plsc.VectorSubcoreMesh(core_axis_name="c", subcore_axis_name="s")

    @functools.partial(
        pl.kernel,
        mesh=mesh,
        out_type=jax.ShapeDtypeStruct((_NC, n, d), jnp.float32),
        scratch_types=[
            pltpu.VMEM((n_chunk, _K), jnp.int32),
            pltpu.VMEM((n_chunk, _K), jnp.int32),
            pltpu.VMEM((n_chunk, _K), jnp.float32),
            pltpu.VMEM((_K, d), jnp.float32),
            pltpu.VMEM_SHARED((n, d), jnp.float32),
            pltpu.SemaphoreType.DMA,
        ],
    )
    def k(h_hbm, src_hbm, dst_hbm, w_hbm, out_hbm,
          srcb, dstb, wb, rows, acc, sem):
        c = lax.axis_index("c")
        s = lax.axis_index("s")
        wid = s * _NC + c

        @pl.loop(0, _K)
        def _(i):
            for j in range(d // _L):
                rows[i, pl.ds(j * _L, _L)] = jnp.zeros((_L,), jnp.float32)

        base_row = s * rows_per_tile
        for p in range(rows_per_tile // _K):
            pltpu.sync_copy(rows, acc.at[pl.ds(base_row + p * _K, _K)])
        ztail = rows_per_tile // _K * _K
        pltpu.sync_copy(rows.at[pl.ds(0, rows_per_tile - ztail)],
                        acc.at[pl.ds(base_row + ztail, rows_per_tile - ztail)])

        @pl.when(s == _NS - 1)
        def _():
            pltpu.sync_copy(rows.at[pl.ds(0, rem)],
                            acc.at[pl.ds(_NS * rows_per_tile, rem)])

        pltpu.sync_copy(src_hbm.at[wid], srcb)
        pltpu.sync_copy(dst_hbm.at[wid], dstb)
        pltpu.sync_copy(w_hbm.at[wid], wb)

        plsc.subcore_barrier()

        @pl.loop(0, n_chunk)
        def _(ci):
            pltpu.async_copy(h_hbm.at[srcb.at[ci]], rows, sem).wait()

            @pl.loop(0, _K // _L)
            def _(g):
                wvec = wb[ci, pl.ds(g * _L, _L)]
                for i in range(_L):
                    wv = jnp.full((_L,), wvec[i], dtype=jnp.float32)
                    for j in range(d // _L):
                        sl = (g * _L + i, pl.ds(j * _L, _L))
                        rows[sl] = rows[sl] * wv

            pltpu.sync_copy(rows, acc.at[dstb.at[ci]], add=True)

        plsc.subcore_barrier()
        pltpu.sync_copy(acc.at[pl.ds(base_row, rows_per_tile)],
                        out_hbm.at[c, pl.ds(base_row, rows_per_tile)])

        @pl.when(s == _NS - 1)
        def _():
            tail = _NS * rows_per_tile
            pltpu.sync_copy(acc.at[pl.ds(tail, rem)],
                            out_hbm.at[c, pl.ds(tail, rem)])

    return k(h, src3, dst3, w3)


def _combine(parts):
    nc, n, d = parts.shape

    def body(p_ref, o_ref):
        o_ref[...] = p_ref[0] + p_ref[1]

    return pl.pallas_call(
        body,
        out_shape=jax.ShapeDtypeStruct((n, d), jnp.float32),
    )(parts)


def _final(x, h1, h2, h3, fw2, mapping):
    n, d = x.shape
    c = mapping.shape[1]

    def body(x_ref, h1_ref, h2_ref, h3_ref, fw_ref, map_ref,
             res_ref, fp_ref):
        fwv = fw_ref[...]
        m = jnp.max(fwv)
        e = jnp.exp(fwv - m)
        fp = e / jnp.sum(e)
        fp_ref[...] = fp

        u = jnp.zeros((n, d), dtype=jnp.float32)
        for i, ref in enumerate((x_ref, h1_ref, h2_ref, h3_ref)):
            f = ref[...]
            mu = jnp.mean(f, axis=0, keepdims=True)
            xc = f - mu
            var = jnp.sum(xc * xc, axis=0, keepdims=True) / (n - 1)
            std = jnp.sqrt(var)
            u = u + fp[0, i] * (xc / (std + 1e-6))
        u = jnp.maximum(u, 0.0)
        logits = jnp.dot(u, map_ref[...], preferred_element_type=jnp.float32)
        lmax = jnp.max(logits, axis=1, keepdims=True)
        ls = logits - lmax
        lse = jnp.log(jnp.sum(jnp.exp(ls), axis=1, keepdims=True))
        res_ref[...] = ls - lse

    return pl.pallas_call(
        body,
        out_shape=(
            jax.ShapeDtypeStruct((n, c), jnp.float32),
            jax.ShapeDtypeStruct((1, 4), jnp.float32),
        ),
    )(x, h1, h2, h3, fw2, mapping)


def kernel(x, edge_index, edge_weight, fw, mapping):
    n, d = x.shape
    e = edge_weight.shape[0]

    epw = -(-e // (_NW * _K)) * _K
    e_pad = epw * _NW
    pad = e_pad - e
    src = jnp.concatenate([edge_index[0], jnp.zeros((pad,), jnp.int32)])
    dst = jnp.concatenate([edge_index[1], jnp.zeros((pad,), jnp.int32)])
    w = jnp.concatenate([edge_weight, jnp.zeros((pad,), jnp.float32)])
    n_chunk = epw // _K
    src3 = src.reshape(_NW, n_chunk, _K)
    dst3 = dst.reshape(_NW, n_chunk, _K)
    w3 = w.reshape(_NW, n_chunk, _K)

    h1 = _combine(_propagate(x, src3, dst3, w3, n_chunk))
    h2 = _combine(_propagate(h1, src3, dst3, w3, n_chunk))
    h3 = _combine(_propagate(h2, src3, dst3, w3, n_chunk))

    res, fp2 = _final(x, h1, h2, h3, fw.reshape(1, 4), mapping)
    return res, fp2.reshape(4), 0
